# baseline (speedup 1.0000x reference)
import jax
import jax.numpy as jnp
from jax.experimental import pallas as pl
from jax.experimental.pallas import tpu as pltpu

D_MODEL = 1024
BS = 512


def _posenc_kernel(t_ref, x_ref, pe_ref, te_ref, o_ref):
    o_ref[...] = x_ref[...] + pe_ref[...] + te_ref[...]


def kernel(x, t, pe, te):
    B, S, D = x.shape
    te2 = te.reshape(te.shape[1], 1, D)
    grid = (S // BS, B)
    out = pl.pallas_call(
        _posenc_kernel,
        grid_spec=pltpu.PrefetchScalarGridSpec(
            num_scalar_prefetch=1,
            grid=grid,
            in_specs=[
                pl.BlockSpec((1, BS, D), lambda j, b, t_ref: (b, j, 0)),
                pl.BlockSpec((1, BS, D), lambda j, b, t_ref: (0, j, 0)),
                pl.BlockSpec((1, 1, D), lambda j, b, t_ref: (t_ref[b], 0, 0)),
            ],
            out_specs=pl.BlockSpec((1, BS, D), lambda j, b, t_ref: (b, j, 0)),
        ),
        out_shape=jax.ShapeDtypeStruct((B, S, D), x.dtype),
    )(t, x, pe, te2)
    return out
